# baseline (device time: 159191 ns/iter reference)
import jax
import jax.numpy as jnp
from jax import lax
from jax.experimental import pallas as pl
from jax.experimental.pallas import tpu as pltpu


def kernel(x, A, B, C):
    Bb, S, D = x.shape
    N = A.shape[1]
    CORR_STEPS = S

    def body(x_ref, a_ref, b_ref, c_ref, out_ref,
             h_send, h_recv, send_sem, recv_sem):
        my_x = lax.axis_index("x")
        my_y = lax.axis_index("y")

        barrier = pltpu.get_barrier_semaphore()
        pl.semaphore_signal(
            barrier, inc=1,
            device_id=(1 - my_x, my_y),
            device_id_type=pl.DeviceIdType.MESH,
        )
        pl.semaphore_wait(barrier, 1)

        dA = jnp.exp(a_ref[...]).T[None]

        def step(t, h):
            xt = x_ref[:, t, :]
            bt = b_ref[:, t, :]
            ct = c_ref[:, t, :]
            h = h * dA + bt[:, :, None] * xt[:, None, :]
            out_ref[:, t, :] = jnp.sum(h * ct[:, :, None], axis=1)
            return h

        h_final = lax.fori_loop(
            0, S, step, jnp.zeros((Bb, N, D), jnp.float32)
        )

        h_send[...] = h_final
        rdma = pltpu.make_async_remote_copy(
            src_ref=h_send, dst_ref=h_recv,
            send_sem=send_sem, recv_sem=recv_sem,
            device_id=(1 - my_x, my_y),
            device_id_type=pl.DeviceIdType.MESH,
        )

        @pl.when(my_x == 0)
        def _():
            rdma.start()
            rdma.wait_send()

        @pl.when(my_x == 1)
        def _():
            rdma.wait_recv()

            def corr(t, g):
                g = g * dA[0]
                ct = c_ref[:, t, :]
                out_ref[:, t, :] = out_ref[:, t, :] + jnp.sum(
                    g * ct[:, :, None], axis=1
                )
                return g

            lax.fori_loop(0, CORR_STEPS, corr, h_recv[...])

    return pl.pallas_call(
        body,
        out_shape=jax.ShapeDtypeStruct((Bb, S, D), jnp.float32),
        in_specs=[pl.BlockSpec(memory_space=pltpu.VMEM)] * 4,
        out_specs=pl.BlockSpec(memory_space=pltpu.VMEM),
        scratch_shapes=[
            pltpu.VMEM((Bb, N, D), jnp.float32),
            pltpu.VMEM((Bb, N, D), jnp.float32),
            pltpu.SemaphoreType.DMA,
            pltpu.SemaphoreType.DMA,
        ],
        compiler_params=pltpu.CompilerParams(collective_id=0),
    )(x, A, B, C)
